# baseline (speedup 1.0000x reference)
import functools

import jax
import jax.numpy as jnp
from jax import lax
from jax.experimental import pallas as pl
from jax.experimental.pallas import tpu as pltpu

DIM = 1024
HEADS = 16
DH = 64
BUCKET = 64
NH = 2
DEPTH = 2
FF_MULT = 4
SEQ = 4096
KEY_LEN = 1024

_INTERPRET = False


def _mm_kernel(x_ref, w_ref, b_ref, *rest, ln, ln_limit, act, n_res, heads_out,
               bm, cast_bf16):
    idx = 0
    if ln:
        g_ref, bb_ref = rest[idx], rest[idx + 1]
        idx += 2
    res_refs = rest[idx:idx + n_res]
    out_ref = rest[idx + n_res]
    x = x_ref[...]
    if ln:
        m = jnp.mean(x, axis=-1, keepdims=True)
        v = jnp.mean((x - m) ** 2, axis=-1, keepdims=True)
        xn = (x - m) / jnp.sqrt(v + 1e-5) * g_ref[...] + bb_ref[...]
        if ln_limit is not None:
            row0 = pl.program_id(0) * bm
            x = jnp.where(row0 < ln_limit, xn, x)
        else:
            x = xn
    w = w_ref[...]
    if cast_bf16:
        x = x.astype(jnp.bfloat16)
        w = w.astype(jnp.bfloat16)
    acc = jnp.dot(x, w, preferred_element_type=jnp.float32)
    acc = acc + b_ref[...]
    if act == "gelu":
        acc = jax.nn.gelu(acc)
    for r in res_refs:
        acc = acc + r[...]
    if heads_out:
        nh = acc.shape[-1] // DH
        for h in range(nh):
            out_ref[h] = acc[:, h * DH:(h + 1) * DH]
    else:
        out_ref[...] = acc


def _mm(x, w, bias, *, ln=None, ln_limit=None, act=None, residuals=(),
        heads_out=False, bm=512, bn=512, cast_bf16=False):
    M, K = x.shape
    N = w.shape[1]
    if heads_out:
        bn = 128
    grid = (M // bm, N // bn)
    in_specs = [
        pl.BlockSpec((bm, K), lambda i, j: (i, 0)),
        pl.BlockSpec((K, bn), lambda i, j: (0, j)),
        pl.BlockSpec((1, bn), lambda i, j: (0, j)),
    ]
    args = [x, w, bias.reshape(1, N)]
    if ln is not None:
        g, b = ln
        in_specs += [pl.BlockSpec((1, K), lambda i, j: (0, 0))] * 2
        args += [g.reshape(1, K), b.reshape(1, K)]
    for r in residuals:
        in_specs.append(pl.BlockSpec((bm, bn), lambda i, j: (i, j)))
        args.append(r)
    if heads_out:
        out_shape = jax.ShapeDtypeStruct((N // DH, M, DH), jnp.float32)
        out_spec = pl.BlockSpec((bn // DH, bm, DH),
                                lambda i, j: (j, i, 0))
    else:
        out_shape = jax.ShapeDtypeStruct((M, N), jnp.float32)
        out_spec = pl.BlockSpec((bm, bn), lambda i, j: (i, j))
    fn = functools.partial(_mm_kernel, ln=ln is not None, ln_limit=ln_limit,
                           act=act, n_res=len(residuals),
                           heads_out=heads_out, bm=bm, cast_bf16=cast_bf16)
    return pl.pallas_call(
        fn, grid=grid, in_specs=in_specs, out_specs=out_spec,
        out_shape=out_shape, interpret=_INTERPRET)(*args)


def _bucket_kernel(qk_ref, rot_ref, out_ref, *, nb):
    R = nb // 2
    rotated = jnp.dot(qk_ref[0].astype(jnp.bfloat16),
                      rot_ref[...].astype(jnp.bfloat16),
                      preferred_element_type=jnp.float32)
    for r in range(NH):
        sl = rotated[:, r * R:(r + 1) * R]
        cat = jnp.concatenate([sl, -sl], axis=-1)
        am = jnp.argmax(cat, axis=-1).astype(jnp.int32) + r * nb
        out_ref[0, r] = am[:, None]


def _bucketize(qk_heads, rot, T):
    nb = T // BUCKET
    BT = 512
    grid = (HEADS, T // BT)
    rot_flat = rot.reshape(DH, NH * (nb // 2))
    out = pl.pallas_call(
        functools.partial(_bucket_kernel, nb=nb),
        grid=grid,
        in_specs=[
            pl.BlockSpec((1, BT, DH), lambda h, s: (h, s, 0)),
            pl.BlockSpec((DH, NH * (nb // 2)), lambda h, s: (0, 0)),
        ],
        out_specs=pl.BlockSpec((1, NH, BT, 1), lambda h, s: (h, 0, s, 0)),
        out_shape=jax.ShapeDtypeStruct((HEADS, NH, T, 1), jnp.int32),
        interpret=_INTERPRET)(qk_heads, rot_flat)
    N = NH * T
    return out.reshape(HEADS, N // 128, 128)


def _pos_kernel(bkt_ref, pos_ref, posg_ref, *, nblk, nbp, n_items):
    h = pl.program_id(0)
    sub_iota = lax.broadcasted_iota(jnp.int32, (nbp, 128), 0)
    tri_items = (lax.broadcasted_iota(jnp.int32, (128, 128), 0)
                 < lax.broadcasted_iota(jnp.int32, (128, 128), 1)
                 ).astype(jnp.bfloat16)
    tri_b = (lax.broadcasted_iota(jnp.int32, (nbp, nbp), 1)
             < lax.broadcasted_iota(jnp.int32, (nbp, nbp), 0)
             ).astype(jnp.float32)

    def oh_block(k):
        seg = bkt_ref[0, k, :].reshape(1, 128)
        return (sub_iota == seg).astype(jnp.float32)

    def body1(k, running):
        return running + jnp.sum(oh_block(k), axis=1, keepdims=True)

    total = lax.fori_loop(0, nblk, body1, jnp.zeros((nbp, 1), jnp.float32))
    starts = jnp.dot(tri_b, total, preferred_element_type=jnp.float32)

    def body2(k, running):
        oh = oh_block(k)
        within = jnp.dot(oh.astype(jnp.bfloat16), tri_items,
                         preferred_element_type=jnp.float32)
        base = starts + running
        pos_seg = jnp.sum(oh * (within + base), axis=0, keepdims=True)
        pos_i = pos_seg.astype(jnp.int32)
        pos_ref[0, k, :] = pos_i[0]
        posg_ref[0, k, :] = pos_i[0] + h * n_items
        return running + jnp.sum(oh, axis=1, keepdims=True)

    lax.fori_loop(0, nblk, body2, jnp.zeros((nbp, 1), jnp.float32))


def _sort_positions(bkt, T):
    N = NH * T
    nblk = N // 128
    nbt = NH * (T // BUCKET)
    nbp = 128 if nbt <= 128 else 256
    out_sh = jax.ShapeDtypeStruct((HEADS, nblk, 128), jnp.int32)
    pos, posg = pl.pallas_call(
        functools.partial(_pos_kernel, nblk=nblk, nbp=nbp, n_items=N),
        grid=(HEADS,),
        in_specs=[pl.BlockSpec((1, nblk, 128), lambda h: (h, 0, 0))],
        out_specs=[pl.BlockSpec((1, nblk, 128), lambda h: (h, 0, 0))] * 2,
        out_shape=[out_sh, out_sh],
        interpret=_INTERPRET)(bkt)
    return pos, posg


def _attn_kernel(qk_ref, qkp_ref, v_ref, vp_ref, stc_ref, str_ref, strp_ref,
                 so_ref, sl_ref, *, causal, cb):
    rows = cb * BUCKET
    cur_qk = qk_ref[0]
    cur_v = v_ref[0]
    prev_qk = jnp.concatenate(
        [qkp_ref[0, rows - BUCKET:rows, :], cur_qk[:rows - BUCKET]], axis=0)
    prev_v = jnp.concatenate(
        [vp_ref[0, rows - BUCKET:rows, :], cur_v[:rows - BUCKET]], axis=0)
    scale = DH ** -0.5
    st_rows = str_ref[0, 0]
    stp_rows = strp_ref[0, 0]
    for j in range(cb):
        sl = slice(j * BUCKET, (j + 1) * BUCKET)
        bq = cur_qk[sl]
        kcat = jnp.concatenate([cur_qk[sl], prev_qk[sl]], axis=0)
        kn = kcat / (jnp.sqrt(jnp.sum(kcat * kcat, axis=-1, keepdims=True))
                     + 1e-9)
        dots = lax.dot_general(bq, kn, (((1,), (1,)), ((), ())),
                               preferred_element_type=jnp.float32) * scale
        r, c0 = j // 2, (j % 2) * BUCKET
        kt = jnp.concatenate(
            [st_rows[r:r + 1, c0:c0 + BUCKET],
             stp_rows[r:r + 1, c0:c0 + BUCKET]], axis=1)
        qt = stc_ref[0, sl, :]
        if causal:
            dots = jnp.where(qt < kt, -1e9, dots)
        dots = jnp.where(qt == kt, dots - 1e5, dots)
        m = jnp.max(dots, axis=-1, keepdims=True)
        lse = m + jnp.log(jnp.sum(jnp.exp(dots - m), axis=-1, keepdims=True))
        probs = jnp.exp(dots - lse)
        vcat = jnp.concatenate([cur_v[sl], prev_v[sl]], axis=0)
        bo = jnp.dot(probs, vcat, preferred_element_type=jnp.float32)
        so_ref[0, sl, :] = bo
        sl_ref[0, sl, :] = lse


def _bucket_attention(sqk, sv, st, T, causal):
    N = NH * T
    CB = 8
    rows = CB * BUCKET
    nc = N // rows
    st_col = st.reshape(HEADS, N, 1)
    rb = rows // 128
    st_row = st.reshape(HEADS, nc, rb, 128)
    stp_row = jnp.roll(st, BUCKET, axis=1).reshape(HEADS, nc, rb, 128)
    grid = (HEADS, nc)
    data_spec = pl.BlockSpec((1, rows, DH), lambda h, c: (h, c, 0))
    prev_spec = pl.BlockSpec((1, rows, DH), lambda h, c: (h, (c - 1) % nc, 0))
    so, slog = pl.pallas_call(
        functools.partial(_attn_kernel, causal=causal, cb=CB),
        grid=grid,
        in_specs=[
            data_spec, prev_spec, data_spec, prev_spec,
            pl.BlockSpec((1, rows, 1), lambda h, c: (h, c, 0)),
            pl.BlockSpec((1, 1, rb, 128), lambda h, c: (h, c, 0, 0)),
            pl.BlockSpec((1, 1, rb, 128), lambda h, c: (h, c, 0, 0)),
        ],
        out_specs=[data_spec,
                   pl.BlockSpec((1, rows, 1), lambda h, c: (h, c, 0))],
        out_shape=[jax.ShapeDtypeStruct((HEADS, N, DH), jnp.float32),
                   jax.ShapeDtypeStruct((HEADS, N, 1), jnp.float32)],
        interpret=_INTERPRET)(sqk, sqk, sv, sv, st_col, st_row, stp_row)
    return so, slog.reshape(HEADS, N)


def _combine_kernel(o_ref, l_ref, out_ref):
    for hh in range(2):
        l = l_ref[hh]
        m = jnp.max(l, axis=0, keepdims=True)
        lse = m + jnp.log(jnp.sum(jnp.exp(l - m), axis=0, keepdims=True))
        w = jnp.exp(l - lse)
        out_ref[:, hh * DH:(hh + 1) * DH] = jnp.sum(o_ref[hh] * w, axis=0)


def _combine(o_u, logits, T, t_out):
    BT = 512
    o4 = o_u.reshape(HEADS, NH, T, DH)
    l4 = logits.reshape(HEADS, NH, T, 1)
    return pl.pallas_call(
        _combine_kernel,
        grid=(HEADS // 2, t_out // BT),
        in_specs=[
            pl.BlockSpec((2, NH, BT, DH), lambda h, s: (h, 0, s, 0)),
            pl.BlockSpec((2, NH, BT, 1), lambda h, s: (h, 0, s, 0)),
        ],
        out_specs=pl.BlockSpec((BT, 2 * DH), lambda h, s: (s, h)),
        out_shape=jax.ShapeDtypeStruct((t_out, DIM), jnp.float32),
        interpret=_INTERPRET)(o4, l4)


def _sort_apply(qk_heads, v_heads, pos, posg, T):
    N = NH * T
    pos2 = pos.reshape(HEADS, N)
    st = (jnp.argsort(pos2, axis=-1) % T).astype(jnp.int32)
    sqk = jnp.take_along_axis(qk_heads, st[..., None], axis=1)
    sv = jnp.take_along_axis(v_heads, st[..., None], axis=1)
    return sqk, sv, st


def _unsort(so, slog, pos, posg, T):
    N = NH * T
    pos2 = pos.reshape(HEADS, N)
    o_u = jnp.take_along_axis(so, pos2[..., None], axis=1)
    logits = jnp.take_along_axis(slog, pos2, axis=1)
    return o_u, logits


def _lsh_attention_block(x2d, res2d, p, causal, keys2d=None):
    t = SEQ
    if keys2d is None:
        xin = x2d
        ln_limit = None
    else:
        xin = jnp.concatenate([x2d, keys2d], axis=0)
        ln_limit = t
    T = xin.shape[0]
    w2 = jnp.concatenate([p["wqk"], p["wv"]], axis=1)
    zeros2 = jnp.zeros((2 * DIM,), jnp.float32)
    heads = _mm(xin, w2, zeros2, ln=(p["ln_g"], p["ln_b"]),
                ln_limit=ln_limit, heads_out=True, cast_bf16=True)
    qk_heads, v_heads = heads[:HEADS], heads[HEADS:]
    bkt = _bucketize(qk_heads, p["rot"], T)
    pos, posg = _sort_positions(bkt, T)
    sqk, sv, st = _sort_apply(qk_heads, v_heads, pos, posg, T)
    so, slog = _bucket_attention(sqk, sv, st, T, causal)
    o_u, logits = _unsort(so, slog, pos, posg, T)
    ctx = _combine(o_u, logits, T, t)
    return _mm(ctx, p["wo"], p["bo"], residuals=(res2d,))


def _feed_forward_block(b2d, a2d, lp, extra_res=None):
    h = _mm(b2d, lp["ff"]["w1"], lp["ff"]["b1"],
            ln=(lp["ln3_g"], lp["ln3_b"]), act="gelu", bn=512)
    res = (a2d,) if extra_res is None else (a2d, extra_res)
    return _mm(h, lp["ff"]["w2"], lp["ff"]["b2"], residuals=res, bm=512)


def kernel(x, keys, params):
    x2 = x[0]
    keys2 = keys[0]
    a, b = x2, x2
    for li, lp in enumerate(params["layers"]):
        sa = dict(lp["self_attn"], ln_g=lp["ln1_g"], ln_b=lp["ln1_b"])
        a_new = _lsh_attention_block(b, a, sa, causal=True)
        a, b = b, a_new
        ca = dict(lp["attn"], ln_g=lp["ln2_g"], ln_b=lp["ln2_b"])
        a_new = _lsh_attention_block(b, a, ca, causal=False, keys2d=keys2)
        a, b = b, a_new
        a_new = _feed_forward_block(b, a, lp)
        a, b = b, a_new
    return (a + b)[None]
